# baseline (speedup 1.0000x reference)
import jax
import jax.numpy as jnp
from jax.experimental import pallas as pl
from jax.experimental.pallas import tpu as pltpu
from functools import partial

E = 8
P = 64
D = 128
TILE = 2400


def _moe_kernel(x_ref, rw_ref, rb_ref, lng_ref, lnb_ref,
                sw_ref, sb_ref, sk_ref, sp_ref,
                tw_ref, tb_ref, tk_ref, tp_ref,
                out_ref):
    x = x_ref[...]

    logits = jax.lax.dot_general(x, rw_ref[...], (((1,), (1,)), ((), ())),
                                 preferred_element_type=jnp.float32)
    logits = logits + rb_ref[...][None, :]
    mu = jnp.mean(logits, axis=-1, keepdims=True)
    var = jnp.mean((logits - mu) ** 2, axis=-1, keepdims=True)
    ln = (logits - mu) * jax.lax.rsqrt(var + 1e-5)
    ln = ln * lng_ref[...][None, :] + lnb_ref[...][None, :]
    pm = jnp.max(ln, axis=-1, keepdims=True)
    ex = jnp.exp(ln - pm)
    probs = ex / jnp.sum(ex, axis=-1, keepdims=True)

    idx = jax.lax.broadcasted_iota(jnp.int32, (TILE, E), 1)
    m1 = jnp.max(probs, axis=-1, keepdims=True)
    i1 = jnp.min(jnp.where(probs == m1, idx, E), axis=-1, keepdims=True)
    oh1 = (idx == i1)
    masked = jnp.where(oh1, -jnp.inf, probs)
    m2 = jnp.max(masked, axis=-1, keepdims=True)
    i2 = jnp.min(jnp.where(masked == m2, idx, E), axis=-1, keepdims=True)
    oh2 = (idx == i2)
    denom = m1 + m2 + 1e-8
    w = (m1 / denom) * oh1.astype(jnp.float32) \
        + (m2 / denom) * oh2.astype(jnp.float32)

    acc = jnp.sum(w, axis=-1, keepdims=True) * x
    for e in range(E):
        we = w[:, e][:, None]

        sq = jax.lax.dot_general(x, sw_ref[e], (((1,), (1,)), ((), ())),
                                 preferred_element_type=jnp.float32)
        sq = jnp.tanh(sq + sb_ref[e][None, :])
        sa = jax.lax.dot_general(sq, sk_ref[e], (((1,), (1,)), ((), ())),
                                 preferred_element_type=jnp.float32)
        sa = jnp.exp(sa - 64.0)
        sa = sa * (1.0 / jnp.sum(sa, axis=-1, keepdims=True))
        so = jnp.dot(sa, sp_ref[e], preferred_element_type=jnp.float32)

        tq = jax.lax.dot_general(x, tw_ref[e], (((1,), (1,)), ((), ())),
                                 preferred_element_type=jnp.float32)
        tq = jnp.tanh(tq + tb_ref[e][None, :])
        ta = jax.lax.dot_general(tq, tk_ref[e], (((1,), (1,)), ((), ())),
                                 preferred_element_type=jnp.float32)
        ta = jnp.exp(ta - 64.0)
        ta = ta * (1.0 / jnp.sum(ta, axis=-1, keepdims=True))
        to = jnp.dot(ta, tp_ref[e], preferred_element_type=jnp.float32)

        acc = acc + we * (so + to)

    out_ref[...] = acc


def _run_moe(x_flat, router_W, router_b, ln_g, ln_b,
             s_pool, s_key, s_W, s_b, t_pool, t_key, t_W, t_b,
             interpret=False):
    M = x_flat.shape[0]
    grid = (M // TILE,)
    full = lambda *shape: pl.BlockSpec(shape, lambda i: (0,) * len(shape))
    return pl.pallas_call(
        _moe_kernel,
        grid=grid,
        in_specs=[
            pl.BlockSpec((TILE, D), lambda i: (i, 0)),
            full(E, D), full(E), full(E), full(E),
            full(E, D, D), full(E, D), full(E, P, D), full(E, P, D),
            full(E, D, D), full(E, D), full(E, P, D), full(E, P, D),
        ],
        out_specs=pl.BlockSpec((TILE, D), lambda i: (i, 0)),
        out_shape=jax.ShapeDtypeStruct((M, D), jnp.float32),
        interpret=interpret,
    )(x_flat, router_W, router_b, ln_g, ln_b,
      s_W, s_b, s_key, s_pool, t_W, t_b, t_key, t_pool)


@partial(jax.jit, static_argnames=("interpret",))
def _impl(maingraph, subgraph, router_W, router_b, ln_g, ln_b,
          s_pool, s_key, s_W, s_b, t_pool, t_key, t_W, t_b,
          interpret=False):
    B, N, T, Dd, C = subgraph.shape
    args = (router_W, router_b, ln_g, ln_b,
            s_pool, s_key, s_W, s_b, t_pool, t_key, t_W, t_b)

    ym = _run_moe(maingraph.reshape(-1, Dd), *args, interpret=interpret)
    main_out = ym.reshape(B, N, T, Dd)

    subs = []
    for c0 in range(C):
        chunk = subgraph[:, :, :, :, c0]
        xc = chunk.reshape(-1, Dd)
        yc = _run_moe(xc, *args, interpret=interpret)
        subs.append(yc.reshape(B, N, T, Dd, 1))
    sub_out = jnp.concatenate(subs, axis=-1)
    return (main_out, sub_out)


def kernel(maingraph, subgraph, router_W, router_b, ln_g, ln_b,
           s_pool, s_key, s_W, s_b, t_pool, t_key, t_W, t_b):
    return _impl(maingraph, subgraph, router_W, router_b, ln_g, ln_b,
                 s_pool, s_key, s_W, s_b, t_pool, t_key, t_W, t_b)


# speedup vs baseline: 1.0335x; 1.0335x over previous
import jax
import jax.numpy as jnp
from jax.experimental import pallas as pl
from jax.experimental.pallas import tpu as pltpu
from functools import partial

E = 8
P = 64
D = 128
TILE = 1200


def _moe_kernel(x_ref, rw_ref, rb_ref, lng_ref, lnb_ref,
                sw_ref, sb_ref, sk_ref, sp_ref,
                tw_ref, tb_ref, tk_ref, tp_ref,
                out_ref):
    x = x_ref[...]

    logits = jax.lax.dot_general(x, rw_ref[...], (((1,), (1,)), ((), ())),
                                 preferred_element_type=jnp.float32)
    logits = logits + rb_ref[...][None, :]
    mu = jnp.mean(logits, axis=-1, keepdims=True)
    var = jnp.mean((logits - mu) ** 2, axis=-1, keepdims=True)
    ln = (logits - mu) * jax.lax.rsqrt(var + 1e-5)
    ln = ln * lng_ref[...][None, :] + lnb_ref[...][None, :]
    pm = jnp.max(ln, axis=-1, keepdims=True)
    ex = jnp.exp(ln - pm)
    probs = ex / jnp.sum(ex, axis=-1, keepdims=True)

    idx = jax.lax.broadcasted_iota(jnp.int32, (TILE, E), 1)
    m1 = jnp.max(probs, axis=-1, keepdims=True)
    i1 = jnp.min(jnp.where(probs == m1, idx, E), axis=-1, keepdims=True)
    oh1 = (idx == i1)
    masked = jnp.where(oh1, -jnp.inf, probs)
    m2 = jnp.max(masked, axis=-1, keepdims=True)
    i2 = jnp.min(jnp.where(masked == m2, idx, E), axis=-1, keepdims=True)
    oh2 = (idx == i2)
    denom = m1 + m2 + 1e-8
    w = (m1 / denom) * oh1.astype(jnp.float32) \
        + (m2 / denom) * oh2.astype(jnp.float32)

    acc = jnp.sum(w, axis=-1, keepdims=True) * x
    for e in range(E):
        we = w[:, e][:, None]

        sq = jax.lax.dot_general(x, sw_ref[e], (((1,), (1,)), ((), ())),
                                 preferred_element_type=jnp.float32)
        sq = jnp.tanh(sq + sb_ref[e][None, :])
        sa = jax.lax.dot_general(sq, sk_ref[e], (((1,), (1,)), ((), ())),
                                 preferred_element_type=jnp.float32)
        sa = jnp.exp(sa - 64.0)
        sa = sa * (1.0 / jnp.sum(sa, axis=-1, keepdims=True))
        so = jnp.dot(sa, sp_ref[e], preferred_element_type=jnp.float32)

        tq = jax.lax.dot_general(x, tw_ref[e], (((1,), (1,)), ((), ())),
                                 preferred_element_type=jnp.float32)
        tq = jnp.tanh(tq + tb_ref[e][None, :])
        ta = jax.lax.dot_general(tq, tk_ref[e], (((1,), (1,)), ((), ())),
                                 preferred_element_type=jnp.float32)
        ta = jnp.exp(ta - 64.0)
        ta = ta * (1.0 / jnp.sum(ta, axis=-1, keepdims=True))
        to = jnp.dot(ta, tp_ref[e], preferred_element_type=jnp.float32)

        acc = acc + we * (so + to)

    out_ref[...] = acc


def _run_moe(x_flat, router_W, router_b, ln_g, ln_b,
             s_pool, s_key, s_W, s_b, t_pool, t_key, t_W, t_b,
             interpret=False):
    M = x_flat.shape[0]
    grid = (M // TILE,)
    full = lambda *shape: pl.BlockSpec(shape, lambda i: (0,) * len(shape))
    return pl.pallas_call(
        _moe_kernel,
        grid=grid,
        in_specs=[
            pl.BlockSpec((TILE, D), lambda i: (i, 0)),
            full(E, D), full(E), full(E), full(E),
            full(E, D, D), full(E, D), full(E, P, D), full(E, P, D),
            full(E, D, D), full(E, D), full(E, P, D), full(E, P, D),
        ],
        out_specs=pl.BlockSpec((TILE, D), lambda i: (i, 0)),
        out_shape=jax.ShapeDtypeStruct((M, D), jnp.float32),
        interpret=interpret,
    )(x_flat, router_W, router_b, ln_g, ln_b,
      s_W, s_b, s_key, s_pool, t_W, t_b, t_key, t_pool)


@partial(jax.jit, static_argnames=("interpret",))
def _impl(maingraph, subgraph, router_W, router_b, ln_g, ln_b,
          s_pool, s_key, s_W, s_b, t_pool, t_key, t_W, t_b,
          interpret=False):
    B, N, T, Dd, C = subgraph.shape
    args = (router_W, router_b, ln_g, ln_b,
            s_pool, s_key, s_W, s_b, t_pool, t_key, t_W, t_b)

    ym = _run_moe(maingraph.reshape(-1, Dd), *args, interpret=interpret)
    main_out = ym.reshape(B, N, T, Dd)

    subs = []
    for c0 in range(0, C, 2):
        chunk = subgraph[:, :, :, :, c0:c0 + 2]
        xc = jnp.transpose(chunk, (0, 4, 1, 2, 3)).reshape(-1, Dd)
        yc = _run_moe(xc, *args, interpret=interpret)
        yc = yc.reshape(B, 2, N, T, Dd)
        subs.append(jnp.transpose(yc, (0, 2, 3, 4, 1)))
    sub_out = jnp.concatenate(subs, axis=-1)
    return (main_out, sub_out)


def kernel(maingraph, subgraph, router_W, router_b, ln_g, ln_b,
           s_pool, s_key, s_W, s_b, t_pool, t_key, t_W, t_b):
    return _impl(maingraph, subgraph, router_W, router_b, ln_g, ln_b,
                 s_pool, s_key, s_W, s_b, t_pool, t_key, t_W, t_b)


# speedup vs baseline: 1.2241x; 1.1844x over previous
import jax
import jax.numpy as jnp
from jax.experimental import pallas as pl
from jax.experimental.pallas import tpu as pltpu
from functools import partial

E = 8
P = 64
D = 128
TILE = 2400


def _moe_kernel(x_ref, rw_ref, rb_ref, lng_ref, lnb_ref,
                sw_ref, sb_ref, sk_ref, sp_ref,
                tw_ref, tb_ref, tk_ref, tp_ref,
                out_ref):
    x = x_ref[...]

    lt = jax.lax.dot_general(rw_ref[...], x, (((1,), (1,)), ((), ())),
                             preferred_element_type=jnp.float32)
    lt = lt + rb_ref[...][:, None]
    mu = jnp.mean(lt, axis=0, keepdims=True)
    var = jnp.mean((lt - mu) ** 2, axis=0, keepdims=True)
    ln = (lt - mu) * jax.lax.rsqrt(var + 1e-5)
    ln = ln * lng_ref[...][:, None] + lnb_ref[...][:, None]
    pm = jnp.max(ln, axis=0, keepdims=True)
    ex = jnp.exp(ln - pm)
    probs = ex * (1.0 / jnp.sum(ex, axis=0, keepdims=True))

    idx = jax.lax.broadcasted_iota(jnp.int32, (E, TILE), 0)
    m1 = jnp.max(probs, axis=0, keepdims=True)
    i1 = jnp.min(jnp.where(probs == m1, idx, E), axis=0, keepdims=True)
    oh1 = (idx == i1)
    masked = jnp.where(oh1, -jnp.inf, probs)
    m2 = jnp.max(masked, axis=0, keepdims=True)
    i2 = jnp.min(jnp.where(masked == m2, idx, E), axis=0, keepdims=True)
    oh2 = (idx == i2)
    denom = m1 + m2 + 1e-8
    wt = (m1 / denom) * oh1.astype(jnp.float32) \
        + (m2 / denom) * oh2.astype(jnp.float32)
    w = wt.T

    acc = jnp.sum(w, axis=-1, keepdims=True) * x
    for e in range(E):
        we = w[:, e][:, None]

        sq = jax.lax.dot_general(x, sw_ref[e], (((1,), (1,)), ((), ())),
                                 preferred_element_type=jnp.float32)
        sq = jnp.tanh(sq + sb_ref[e][None, :])
        sa = jax.lax.dot_general(sq, sk_ref[e], (((1,), (1,)), ((), ())),
                                 preferred_element_type=jnp.float32)
        sa = jnp.exp(sa - 64.0)
        sa = sa * (1.0 / jnp.sum(sa, axis=-1, keepdims=True))
        so = jnp.dot(sa, sp_ref[e], preferred_element_type=jnp.float32)

        tq = jax.lax.dot_general(x, tw_ref[e], (((1,), (1,)), ((), ())),
                                 preferred_element_type=jnp.float32)
        tq = jnp.tanh(tq + tb_ref[e][None, :])
        ta = jax.lax.dot_general(tq, tk_ref[e], (((1,), (1,)), ((), ())),
                                 preferred_element_type=jnp.float32)
        ta = jnp.exp(ta - 64.0)
        ta = ta * (1.0 / jnp.sum(ta, axis=-1, keepdims=True))
        to = jnp.dot(ta, tp_ref[e], preferred_element_type=jnp.float32)

        acc = acc + we * (so + to)

    out_ref[...] = acc


def _run_moe(x_flat, router_W, router_b, ln_g, ln_b,
             s_pool, s_key, s_W, s_b, t_pool, t_key, t_W, t_b,
             interpret=False):
    M = x_flat.shape[0]
    grid = (M // TILE,)
    full = lambda *shape: pl.BlockSpec(shape, lambda i: (0,) * len(shape))
    return pl.pallas_call(
        _moe_kernel,
        grid=grid,
        in_specs=[
            pl.BlockSpec((TILE, D), lambda i: (i, 0)),
            full(E, D), full(E), full(E), full(E),
            full(E, D, D), full(E, D), full(E, P, D), full(E, P, D),
            full(E, D, D), full(E, D), full(E, P, D), full(E, P, D),
        ],
        out_specs=pl.BlockSpec((TILE, D), lambda i: (i, 0)),
        out_shape=jax.ShapeDtypeStruct((M, D), jnp.float32),
        interpret=interpret,
    )(x_flat, router_W, router_b, ln_g, ln_b,
      s_W, s_b, s_key, s_pool, t_W, t_b, t_key, t_pool)


@partial(jax.jit, static_argnames=("interpret",))
def _impl(maingraph, subgraph, router_W, router_b, ln_g, ln_b,
          s_pool, s_key, s_W, s_b, t_pool, t_key, t_W, t_b,
          interpret=False):
    B, N, T, Dd, C = subgraph.shape
    args = (router_W, router_b, ln_g, ln_b,
            s_pool, s_key, s_W, s_b, t_pool, t_key, t_W, t_b)

    ym = _run_moe(maingraph.reshape(-1, Dd), *args, interpret=interpret)
    main_out = ym.reshape(B, N, T, Dd)

    subs = []
    for c0 in range(0, C, 2):
        chunk = subgraph[:, :, :, :, c0:c0 + 2]
        xc = jnp.transpose(chunk, (0, 4, 1, 2, 3)).reshape(-1, Dd)
        yc = _run_moe(xc, *args, interpret=interpret)
        yc = yc.reshape(B, 2, N, T, Dd)
        subs.append(jnp.transpose(yc, (0, 2, 3, 4, 1)))
    sub_out = jnp.concatenate(subs, axis=-1)
    return (main_out, sub_out)


def kernel(maingraph, subgraph, router_W, router_b, ln_g, ln_b,
           s_pool, s_key, s_W, s_b, t_pool, t_key, t_W, t_b):
    return _impl(maingraph, subgraph, router_W, router_b, ln_g, ln_b,
                 s_pool, s_key, s_W, s_b, t_pool, t_key, t_W, t_b)
